# baseline (speedup 1.0000x reference)
import functools

import jax
import jax.numpy as jnp
from jax import lax
from jax.experimental import pallas as pl
from jax.experimental.pallas import tpu as pltpu
from jax.experimental.pallas import tpu_sc as plsc

NUM_ENTITIES = 1000000
EMBED_DIM = 64
BATCH = 16384

NC, NS, L = 2, 16, 16
NW = NC * NS
B_PER_W = BATCH // NW
CH = 128
NCHUNK = B_PER_W // CH


def _neg_sqrt(s):
    i = plsc.bitcast(s, jnp.int32)
    t = plsc.bitcast(jnp.int32(0x5F3759DF) - (i >> 1), jnp.float32)
    half_s = s * 0.5
    for _ in range(3):
        t = t * (1.5 - half_s * t * t)
    return -(s * t)


def _tile_body(x_hbm, y_hbm, table_hbm, out_hbm, idx_v, buf, out_v,
               sem0, sem1):
    wid = lax.axis_index("s") * NC + lax.axis_index("c")
    base = wid * B_PER_W
    sems = [sem0, sem1]

    pltpu.sync_copy(x_hbm.at[pl.ds(base, B_PER_W)],
                    idx_v.at[pl.ds(0, B_PER_W)])
    pltpu.sync_copy(y_hbm.at[pl.ds(base, B_PER_W)],
                    idx_v.at[pl.ds(B_PER_W, B_PER_W)])

    def xcopy(c):
        return pltpu.make_async_copy(
            table_hbm.at[idx_v.at[pl.ds(c * CH, CH)]],
            buf.at[c % 2, pl.ds(0, CH)], sems[c % 2])

    def ycopy(c):
        return pltpu.make_async_copy(
            table_hbm.at[idx_v.at[pl.ds(B_PER_W + c * CH, CH)]],
            buf.at[c % 2, pl.ds(CH, CH)], sems[c % 2])

    def fire(c):
        xcopy(c).start()
        ycopy(c).start()

    def drain(c):
        xcopy(c).wait()
        ycopy(c).wait()

    lane = lax.iota(jnp.int32, L)

    def compute(c):
        p = c % 2

        def grp_body(j, carry):
            res = jnp.zeros((L,), jnp.float32)
            for l in range(L):
                e = j * L + l
                sq = jnp.zeros((L,), jnp.float32)
                for k in range(EMBED_DIM // L):
                    xv = buf[p, e, pl.ds(k * L, L)]
                    yv = buf[p, CH + e, pl.ds(k * L, L)]
                    df = xv - yv
                    sq = sq + df * df
                s = jnp.sum(sq)
                res = jnp.where(lane == l, s, res)
            out_v[pl.ds(c * CH + j * L, L)] = _neg_sqrt(res)
            return carry
        lax.fori_loop(0, CH // L, grp_body, 0)

    fire(0)
    for c in range(NCHUNK):
        drain(c)
        if c + 1 < NCHUNK:
            fire(c + 1)
        compute(c)

    pltpu.sync_copy(out_v, out_hbm.at[pl.ds(base, B_PER_W)])


@functools.partial(jax.jit, static_argnames=())
def kernel(x, y, entity_embeddings):
    mesh = plsc.VectorSubcoreMesh(core_axis_name="c", subcore_axis_name="s")
    run = pl.kernel(
        _tile_body,
        out_type=jax.ShapeDtypeStruct((BATCH,), jnp.float32),
        mesh=mesh,
        scratch_types=[
            pltpu.VMEM((2 * B_PER_W,), jnp.int32),
            pltpu.VMEM((2, 2 * CH, EMBED_DIM), jnp.float32),
            pltpu.VMEM((B_PER_W,), jnp.float32),
            pltpu.SemaphoreType.DMA,
            pltpu.SemaphoreType.DMA,
        ],
        compiler_params=pltpu.CompilerParams(
            needs_layout_passes=False, use_tc_tiling_on_sc=False),
    )
    return run(x.astype(jnp.int32), y.astype(jnp.int32), entity_embeddings)


# speedup vs baseline: 1.1082x; 1.1082x over previous
import functools

import jax
import jax.numpy as jnp
from jax import lax
from jax.experimental import pallas as pl
from jax.experimental.pallas import tpu as pltpu
from jax.experimental.pallas import tpu_sc as plsc

NUM_ENTITIES = 1000000
EMBED_DIM = 64
PAD_DIM = 128
BATCH = 16384

NC, NS, L = 2, 16, 16
NW = NC * NS
B_PER_W = BATCH // NW
CH = 128
NCHUNK = B_PER_W // CH


def _neg_sqrt(s):
    i = plsc.bitcast(s, jnp.int32)
    t = plsc.bitcast(jnp.int32(0x5F3759DF) - (i >> 1), jnp.float32)
    half_s = s * 0.5
    for _ in range(3):
        t = t * (1.5 - half_s * t * t)
    return -(s * t)


def _tile_body(x_hbm, y_hbm, table_hbm, out_hbm, idx_v, buf, out_v,
               sem0, sem1):
    wid = lax.axis_index("s") * NC + lax.axis_index("c")
    base = wid * B_PER_W
    sems = [sem0, sem1]

    pltpu.sync_copy(x_hbm.at[pl.ds(base, B_PER_W)],
                    idx_v.at[pl.ds(0, B_PER_W)])
    pltpu.sync_copy(y_hbm.at[pl.ds(base, B_PER_W)],
                    idx_v.at[pl.ds(B_PER_W, B_PER_W)])

    def xcopy(c):
        return pltpu.make_async_copy(
            table_hbm.at[idx_v.at[pl.ds(c * CH, CH)]],
            buf.at[c % 2, pl.ds(0, CH)], sems[c % 2])

    def ycopy(c):
        return pltpu.make_async_copy(
            table_hbm.at[idx_v.at[pl.ds(B_PER_W + c * CH, CH)]],
            buf.at[c % 2, pl.ds(CH, CH)], sems[c % 2])

    def fire(c):
        xcopy(c).start()
        ycopy(c).start()

    def drain(c):
        xcopy(c).wait()
        ycopy(c).wait()

    lane = lax.iota(jnp.int32, L)

    def compute(c):
        p = c % 2

        def grp_body(j, carry):
            res = jnp.zeros((L,), jnp.float32)
            for l in range(L):
                e = j * L + l
                sq = jnp.zeros((L,), jnp.float32)
                for k in range(EMBED_DIM // L):
                    xv = buf[p, e, pl.ds(k * L, L)]
                    yv = buf[p, CH + e, pl.ds(k * L, L)]
                    df = xv - yv
                    sq = sq + df * df
                s = jnp.sum(sq)
                res = jnp.where(lane == l, s, res)
            out_v[pl.ds(c * CH + j * L, L)] = _neg_sqrt(res)
            return carry
        lax.fori_loop(0, CH // L, grp_body, 0)

    fire(0)
    for c in range(NCHUNK):
        drain(c)
        if c + 1 < NCHUNK:
            fire(c + 1)
        compute(c)

    pltpu.sync_copy(out_v, out_hbm.at[pl.ds(base, B_PER_W)])


@functools.partial(jax.jit, static_argnames=())
def kernel(x, y, entity_embeddings):
    mesh = plsc.VectorSubcoreMesh(core_axis_name="c", subcore_axis_name="s")
    run = pl.kernel(
        _tile_body,
        out_type=jax.ShapeDtypeStruct((BATCH,), jnp.float32),
        mesh=mesh,
        scratch_types=[
            pltpu.VMEM((2 * B_PER_W,), jnp.int32),
            pltpu.VMEM((2, 2 * CH, PAD_DIM), jnp.float32),
            pltpu.VMEM((B_PER_W,), jnp.float32),
            pltpu.SemaphoreType.DMA,
            pltpu.SemaphoreType.DMA,
        ],
        compiler_params=pltpu.CompilerParams(needs_layout_passes=False),
    )
    table_pad = jnp.pad(entity_embeddings, ((0, 0), (0, PAD_DIM - EMBED_DIM)))
    return run(x.astype(jnp.int32), y.astype(jnp.int32), table_pad)


# speedup vs baseline: 1.6856x; 1.5210x over previous
import functools

import jax
import jax.numpy as jnp
from jax import lax
from jax.experimental import pallas as pl
from jax.experimental.pallas import tpu as pltpu
from jax.experimental.pallas import tpu_sc as plsc

NUM_ENTITIES = 1000000
EMBED_DIM = 64
BATCH = 16384

NC, NS, L = 2, 16, 16
NW = NC * NS
B_PER_W = BATCH // NW
CH = 64
NCHUNK = B_PER_W // CH


def _neg_sqrt(s):
    i = plsc.bitcast(s, jnp.int32)
    t = plsc.bitcast(jnp.int32(0x5F3759DF) - (i >> 1), jnp.float32)
    half_s = s * 0.5
    for _ in range(3):
        t = t * (1.5 - half_s * t * t)
    return -(s * t)


def _tile_body(x_hbm, y_hbm, table_hbm, out_hbm, idx_v, buf, out_v,
               sem0, sem1):
    wid = lax.axis_index("s") * NC + lax.axis_index("c")
    base = wid * B_PER_W
    sems = [sem0, sem1]

    pltpu.sync_copy(x_hbm.at[pl.ds(base, B_PER_W)],
                    idx_v.at[pl.ds(0, B_PER_W)])
    pltpu.sync_copy(y_hbm.at[pl.ds(base, B_PER_W)],
                    idx_v.at[pl.ds(B_PER_W, B_PER_W)])

    def fire(c):
        p = c % 2

        def body(g, carry):
            xiv = idx_v[pl.ds(c * CH + g * L, L)]
            yiv = idx_v[pl.ds(B_PER_W + c * CH + g * L, L)]
            for l in range(L):
                e = g * L + l
                pltpu.make_async_copy(table_hbm.at[xiv[l]],
                                      buf.at[p, e], sems[p]).start()
                pltpu.make_async_copy(table_hbm.at[yiv[l]],
                                      buf.at[p, CH + e], sems[p]).start()
            return carry
        lax.fori_loop(0, CH // L, body, 0)

    def drain(c):
        p = c % 2

        def body(e, carry):
            pltpu.make_async_copy(table_hbm.at[0],
                                  buf.at[p, 0], sems[p]).wait()
            pltpu.make_async_copy(table_hbm.at[0],
                                  buf.at[p, CH], sems[p]).wait()
            return carry
        lax.fori_loop(0, CH, body, 0)

    lane = lax.iota(jnp.int32, L)

    def compute(c):
        p = c % 2

        def grp_body(j, carry):
            res = jnp.zeros((L,), jnp.float32)
            for l in range(L):
                e = j * L + l
                sq = jnp.zeros((L,), jnp.float32)
                for k in range(EMBED_DIM // L):
                    xv = buf[p, e, pl.ds(k * L, L)]
                    yv = buf[p, CH + e, pl.ds(k * L, L)]
                    df = xv - yv
                    sq = sq + df * df
                s = jnp.sum(sq)
                res = jnp.where(lane == l, s, res)
            out_v[pl.ds(c * CH + j * L, L)] = _neg_sqrt(res)
            return carry
        lax.fori_loop(0, CH // L, grp_body, 0)

    fire(0)
    for c in range(NCHUNK):
        drain(c)
        if c + 1 < NCHUNK:
            fire(c + 1)
        compute(c)

    pltpu.sync_copy(out_v, out_hbm.at[pl.ds(base, B_PER_W)])


@functools.partial(jax.jit, static_argnames=())
def kernel(x, y, entity_embeddings):
    mesh = plsc.VectorSubcoreMesh(core_axis_name="c", subcore_axis_name="s")
    run = pl.kernel(
        _tile_body,
        out_type=jax.ShapeDtypeStruct((BATCH,), jnp.float32),
        mesh=mesh,
        scratch_types=[
            pltpu.VMEM((2 * B_PER_W,), jnp.int32),
            pltpu.VMEM((2, 2 * CH, EMBED_DIM), jnp.float32),
            pltpu.VMEM((B_PER_W,), jnp.float32),
            pltpu.SemaphoreType.DMA,
            pltpu.SemaphoreType.DMA,
        ],
        compiler_params=pltpu.CompilerParams(needs_layout_passes=False),
    )
    return run(x.astype(jnp.int32), y.astype(jnp.int32), entity_embeddings)
